# baseline (speedup 1.0000x reference)
import functools

import jax
import jax.numpy as jnp
from jax import lax
from jax.experimental import pallas as pl
from jax.experimental.pallas import tpu as pltpu
from jax.experimental.pallas import tpu_sc as plsc

L = 16
NC = 2
NS = 16
NW = NC * NS
U = 4
G = 4


def _merge4(a, b):
    z1 = jnp.maximum(a[0], b[3])
    z2 = jnp.maximum(a[1], b[2])
    z3 = jnp.maximum(a[2], b[1])
    z4 = jnp.maximum(a[3], b[0])
    w1 = jnp.maximum(z1, z3); w3 = jnp.minimum(z1, z3)
    w2 = jnp.maximum(z2, z4); w4 = jnp.minimum(z2, z4)
    s1 = jnp.maximum(w1, w2); s2 = jnp.minimum(w1, w2)
    s3 = jnp.maximum(w3, w4); s4 = jnp.minimum(w3, w4)
    return (s1, s2, s3, s4)


def _sc_matcher(B, HW, H, W, rows_per_w, blk):
    nvr = HW // L
    wpr = W // L
    nblk = rows_per_w // blk
    mesh = plsc.VectorSubcoreMesh(core_axis_name="c", subcore_axis_name="s")

    @functools.partial(
        pl.kernel,
        out_type=jax.ShapeDtypeStruct((4 * B * NS * HW,), jnp.float32),
        mesh=mesh,
        compiler_params=pltpu.CompilerParams(needs_layout_passes=False),
        scratch_types=[
            pltpu.VMEM((blk, H, W), jnp.float32),
            pltpu.VMEM((blk, H, W), jnp.float32),
            pltpu.VMEM((4 * HW,), jnp.float32),
            pltpu.VMEM((4 * rows_per_w * L,), jnp.float32),
            pltpu.VMEM((blk * L,), jnp.float32),
            pltpu.VMEM((2 * L,), jnp.float32),
            pltpu.SemaphoreType.DMA,
            pltpu.SemaphoreType.DMA,
        ],
    )
    def body(prev_hbm, init_hbm, wall_hbm,
             out_hbm, rowbuf0, rowbuf1, acc, wbuf, xcbuf, mbuf, sem0, sem1):
        rowbufs = (rowbuf0, rowbuf1)
        wid = lax.axis_index("s") * NC + lax.axis_index("c")
        b = wid // NS
        k = wid % NS
        r0 = wid * rows_per_w
        rb0 = k * rows_per_w
        sems = (sem0, sem1)

        zeros = jnp.zeros((L,), jnp.float32)
        ones = jnp.full((L,), 1.0, jnp.float32)
        neg = jnp.full((L,), -jnp.inf, jnp.float32)
        pos = jnp.full((L,), jnp.inf, jnp.float32)

        def zero_acc(j, _):
            acc[pl.ds(j * L, L)] = zeros
            return 0
        lax.fori_loop(0, 4 * nvr, zero_acc, 0)
        mbuf[pl.ds(0, L)] = zeros
        mbuf[pl.ds(L, L)] = zeros

        rows = B * HW
        for seg in range(4):
            pltpu.sync_copy(
                wall_hbm.at[pl.ds(seg * rows * L + r0 * L, rows_per_w * L)],
                wbuf.at[pl.ds(seg * rows_per_w * L, rows_per_w * L)])

        def start_dma(arr_hbm, bi, p):
            pltpu.async_copy(
                arr_hbm.at[b, pl.ds(rb0 + bi * blk, blk)],
                rowbufs[p], sems[p])

        def wait_dma(p):
            pltpu.make_async_copy(
                prev_hbm.at[0, pl.ds(0, blk)], rowbufs[p], sems[p]).wait()

        start_dma(prev_hbm, 0, 0)
        start_dma(prev_hbm, 1, 1)

        def prev_outer(h, _):
            for p in range(2):
                bi = h * 2 + p
                wait_dma(p)
                rb = rowbufs[p]

                def rowA(rr, _):
                    def p1(s, c):
                        ts0, ts1, ts2, rmin = c
                        tss = [ts0, ts1, ts2]
                        vs = []
                        for u in range(wpr):
                            v = rb[rr, s, pl.ds(u * L, L)]
                            vs.append(v)
                            t1, t2, t3, t4 = tss[u]
                            lo = jnp.minimum(t1, v); t1 = jnp.maximum(t1, v)
                            lo2 = jnp.minimum(t2, lo); t2 = jnp.maximum(t2, lo)
                            lo3 = jnp.minimum(t3, lo2); t3 = jnp.maximum(t3, lo2)
                            t4 = jnp.maximum(t4, lo3)
                            tss[u] = (t1, t2, t3, t4)
                        m01 = jnp.minimum(vs[0], vs[1])
                        rmin = jnp.minimum(rmin, jnp.minimum(m01, vs[2]))
                        return (tss[0], tss[1], tss[2], rmin)

                    t0 = (neg, neg, neg, neg)
                    ts0, ts1, ts2, rmin = lax.fori_loop(
                        0, H, p1, (t0, t0, t0, pos))
                    ts = _merge4(_merge4(ts0, ts1), ts2)

                    def count_eq(m_s):
                        mb = jnp.full((L,), m_s)
                        tot = jnp.float32(0.0)
                        for t in ts:
                            tot = tot + jnp.sum(jnp.where(t == mb, ones, zeros))
                        return tot

                    def next_max(m_s):
                        mb = jnp.full((L,), m_s)
                        cur = neg
                        for t in ts:
                            cur = jnp.maximum(cur, jnp.where(t < mb, t, neg))
                        return jnp.max(cur)

                    m1 = jnp.max(ts[0])
                    c1 = count_eq(m1)
                    m2 = next_max(m1)
                    c2 = count_eq(m2)
                    m3 = next_max(m2)
                    c3 = count_eq(m3)
                    m4 = next_max(m3)
                    four = jnp.float32(4.0)
                    xcut = jnp.where(
                        c1 >= four, m1,
                        jnp.where(c1 + c2 >= four, m2,
                                  jnp.where(c1 + c2 + c3 >= four, m3, m4)))
                    xcbuf[pl.ds(rr * L, L)] = jnp.full((L,), xcut)

                    mnv = jnp.full((L,), jnp.min(rmin))
                    widx = (bi * blk + rr) * L
                    wb = wbuf[pl.ds(widx, L)]
                    wf = wbuf[pl.ds(rows_per_w * L + widx, L)]
                    mbuf[pl.ds(0, L)] = jnp.maximum(mbuf[pl.ds(0, L)], wb * mnv)
                    mbuf[pl.ds(L, L)] = jnp.maximum(mbuf[pl.ds(L, L)], wf * mnv)
                    return 0
                lax.fori_loop(0, blk, rowA, 0)

                for g in range(blk // G):
                    xcs, wbs, wfs = [], [], []
                    for r in range(G):
                        row = g * G + r
                        widx = (bi * blk + row) * L
                        xcs.append(xcbuf[pl.ds(row * L, L)])
                        wbs.append(wbuf[pl.ds(widx, L)])
                        wfs.append(wbuf[pl.ds(rows_per_w * L + widx, L)])

                    def pB(s, _):
                        for cc in range(wpr):
                            j = s * wpr + cc
                            ab = acc[pl.ds(2 * HW + j * L, L)]
                            af = acc[pl.ds(3 * HW + j * L, L)]
                            for r in range(G):
                                v = rb[g * G + r, s, pl.ds(cc * L, L)]
                                xm = jnp.where(v >= xcs[r], v, zeros)
                                ab = jnp.maximum(ab, xm * wbs[r])
                                af = jnp.maximum(af, xm * wfs[r])
                            acc[pl.ds(2 * HW + j * L, L)] = ab
                            acc[pl.ds(3 * HW + j * L, L)] = af
                        return 0
                    lax.fori_loop(0, H, pB, 0)

                @pl.when(bi + 2 < nblk)
                def _():
                    start_dma(prev_hbm, bi + 2, p)
            return 0
        lax.fori_loop(0, nblk // 2, prev_outer, 0)

        start_dma(init_hbm, 0, 0)
        start_dma(init_hbm, 1, 1)

        def init_outer(h, _):
            for p in range(2):
                bi = h * 2 + p
                wait_dma(p)
                rb = rowbufs[p]
                for g in range(blk // G):
                    wbs, wfs = [], []
                    for r in range(G):
                        widx = (bi * blk + g * G + r) * L
                        wbs.append(wbuf[pl.ds(2 * rows_per_w * L + widx, L)])
                        wfs.append(wbuf[pl.ds(3 * rows_per_w * L + widx, L)])

                    def pG(s, _):
                        for cc in range(wpr):
                            j = s * wpr + cc
                            ab = acc[pl.ds(j * L, L)]
                            af = acc[pl.ds(HW + j * L, L)]
                            for r in range(G):
                                v = rb[g * G + r, s, pl.ds(cc * L, L)]
                                ab = jnp.maximum(ab, v * wbs[r])
                                af = jnp.maximum(af, v * wfs[r])
                            acc[pl.ds(j * L, L)] = ab
                            acc[pl.ds(HW + j * L, L)] = af
                        return 0
                    lax.fori_loop(0, H, pG, 0)

                @pl.when(bi + 2 < nblk)
                def _():
                    start_dma(init_hbm, bi + 2, p)
            return 0
        lax.fori_loop(0, nblk // 2, init_outer, 0)

        mbv = jnp.full((L,), jnp.max(mbuf[pl.ds(0, L)]))
        mfv = jnp.full((L,), jnp.max(mbuf[pl.ds(L, L)]))

        def clamp(j, _):
            acc[pl.ds(2 * HW + j * L, L)] = jnp.maximum(
                acc[pl.ds(2 * HW + j * L, L)], mbv)
            acc[pl.ds(3 * HW + j * L, L)] = jnp.maximum(
                acc[pl.ds(3 * HW + j * L, L)], mfv)
            return 0
        lax.fori_loop(0, nvr, clamp, 0)

        for ci in range(4):
            q = b * 4 + ci
            pltpu.sync_copy(acc.at[pl.ds(ci * HW, HW)],
                            out_hbm.at[pl.ds((q * NS + k) * HW, HW)])

    return body


def _sc_reduce(B, HW, H, W):
    nvr = HW // L
    wpr = W // L
    mesh = plsc.VectorSubcoreMesh(core_axis_name="c", subcore_axis_name="s")

    @functools.partial(
        pl.kernel,
        out_type=jax.ShapeDtypeStruct((B, 4, H, W), jnp.float32),
        mesh=mesh,
        compiler_params=pltpu.CompilerParams(needs_layout_passes=False),
        scratch_types=[
            pltpu.VMEM((NS * HW,), jnp.float32),
            pltpu.VMEM((H, W), jnp.float32),
            pltpu.SemaphoreType.DMA,
        ],
    )
    def body(part_hbm, out_hbm, pbuf, obuf, sem):
        wid = lax.axis_index("s") * NC + lax.axis_index("c")

        @pl.when(wid < 4 * B)
        def _():
            q = wid
            bb = q // 4
            ci = q % 4
            pltpu.async_copy(
                part_hbm.at[pl.ds(q * NS * HW, NS * HW)], pbuf, sem).wait()

            def red(s, _):
                for c in range(wpr):
                    j = s * wpr + c
                    m = pbuf[pl.ds(j * L, L)]
                    for kk in range(1, NS):
                        m = jnp.maximum(m, pbuf[pl.ds(kk * HW + j * L, L)])
                    obuf[s, pl.ds(c * L, L)] = m
                return 0
            lax.fori_loop(0, H, red, 0)
            pltpu.sync_copy(obuf, out_hbm.at[bb, ci])

    return body


def kernel(init_sim, prev_sim, init_seg, prev_seg):
    B, HW, H, W = init_sim.shape
    rows = B * HW
    rows_per_w = rows // NW
    blk = 8

    wall = jnp.stack([prev_seg[:, 0].reshape(rows),
                      prev_seg[:, 1].reshape(rows),
                      init_seg[:, 0].reshape(rows),
                      init_seg[:, 1].reshape(rows)], 0)
    wall = jnp.broadcast_to(wall[:, :, None], (4, rows, L)).reshape(4 * rows * L)

    part = _sc_matcher(B, HW, H, W, rows_per_w, blk)(
        prev_sim, init_sim, wall)
    return _sc_reduce(B, HW, H, W)(part)


# speedup vs baseline: 1.2431x; 1.2431x over previous
import functools

import jax
import jax.numpy as jnp
from jax import lax
from jax.experimental import pallas as pl
from jax.experimental.pallas import tpu as pltpu
from jax.experimental.pallas import tpu_sc as plsc

L = 16
NC = 2
NS = 16
NW = NC * NS
U = 4
G = 4


def _merge4(a, b):
    z1 = jnp.maximum(a[0], b[3])
    z2 = jnp.maximum(a[1], b[2])
    z3 = jnp.maximum(a[2], b[1])
    z4 = jnp.maximum(a[3], b[0])
    w1 = jnp.maximum(z1, z3); w3 = jnp.minimum(z1, z3)
    w2 = jnp.maximum(z2, z4); w4 = jnp.minimum(z2, z4)
    s1 = jnp.maximum(w1, w2); s2 = jnp.minimum(w1, w2)
    s3 = jnp.maximum(w3, w4); s4 = jnp.minimum(w3, w4)
    return (s1, s2, s3, s4)


def _sc_matcher(B, HW, rows_per_w, blk):
    nvr = HW // L
    nblk = rows_per_w // blk
    mesh = plsc.VectorSubcoreMesh(core_axis_name="c", subcore_axis_name="s")

    @functools.partial(
        pl.kernel,
        out_type=jax.ShapeDtypeStruct((2 * B * NS * HW,), jnp.float32),
        mesh=mesh,
        compiler_params=pltpu.CompilerParams(needs_layout_passes=False),
        scratch_types=[
            pltpu.VMEM((blk, HW), jnp.float32),
            pltpu.VMEM((blk, HW), jnp.float32),
            pltpu.VMEM((2 * HW,), jnp.float32),
            pltpu.VMEM((2 * rows_per_w * L,), jnp.float32),
            pltpu.VMEM((blk * L,), jnp.float32),
            pltpu.VMEM((2 * L,), jnp.float32),
            pltpu.SemaphoreType.DMA,
            pltpu.SemaphoreType.DMA,
        ],
    )
    def body(prev_hbm, wall_hbm,
             out_hbm, rowbuf0, rowbuf1, acc, wbuf, xcbuf, mbuf, sem0, sem1):
        rowbufs = (rowbuf0, rowbuf1)
        wid = lax.axis_index("s") * NC + lax.axis_index("c")
        b = wid // NS
        k = wid % NS
        r0 = wid * rows_per_w
        rb0 = k * rows_per_w
        sems = (sem0, sem1)
        rows = B * HW

        zeros = jnp.zeros((L,), jnp.float32)
        ones = jnp.full((L,), 1.0, jnp.float32)
        neg = jnp.full((L,), -jnp.inf, jnp.float32)
        pos = jnp.full((L,), jnp.inf, jnp.float32)

        def zero_acc(j, _):
            acc[pl.ds(j * L, L)] = zeros
            return 0
        lax.fori_loop(0, 2 * nvr, zero_acc, 0)
        mbuf[pl.ds(0, L)] = zeros
        mbuf[pl.ds(L, L)] = zeros

        for seg in range(2):
            pltpu.sync_copy(
                wall_hbm.at[pl.ds(seg * rows * L + r0 * L, rows_per_w * L)],
                wbuf.at[pl.ds(seg * rows_per_w * L, rows_per_w * L)])

        def start_dma(bi, p):
            pltpu.async_copy(
                prev_hbm.at[b, pl.ds(rb0 + bi * blk, blk)],
                rowbufs[p], sems[p])

        def wait_dma(p):
            pltpu.make_async_copy(
                prev_hbm.at[0, pl.ds(0, blk)], rowbufs[p], sems[p]).wait()

        start_dma(0, 0)
        start_dma(1, 1)

        def prev_outer(h, _):
            for p in range(2):
                bi = h * 2 + p
                wait_dma(p)
                rb = rowbufs[p]

                def rowA(rr, _):
                    def p1(jj, c):
                        ts0, ts1, ts2, ts3, rmin = c
                        tss = [ts0, ts1, ts2, ts3]
                        vs = []
                        for u in range(U):
                            v = rb[rr, pl.ds((jj * U + u) * L, L)]
                            vs.append(v)
                            t1, t2, t3, t4 = tss[u]
                            lo = jnp.minimum(t1, v); t1 = jnp.maximum(t1, v)
                            lo2 = jnp.minimum(t2, lo); t2 = jnp.maximum(t2, lo)
                            lo3 = jnp.minimum(t3, lo2); t3 = jnp.maximum(t3, lo2)
                            t4 = jnp.maximum(t4, lo3)
                            tss[u] = (t1, t2, t3, t4)
                        m01 = jnp.minimum(vs[0], vs[1])
                        m23 = jnp.minimum(vs[2], vs[3])
                        rmin = jnp.minimum(rmin, jnp.minimum(m01, m23))
                        return (tss[0], tss[1], tss[2], tss[3], rmin)

                    t0 = (neg, neg, neg, neg)
                    ts0, ts1, ts2, ts3, rmin = lax.fori_loop(
                        0, nvr // U, p1, (t0, t0, t0, t0, pos))
                    ts = _merge4(_merge4(ts0, ts1), _merge4(ts2, ts3))

                    def count_eq(m_s):
                        mb = jnp.full((L,), m_s)
                        tot = jnp.float32(0.0)
                        for t in ts:
                            tot = tot + jnp.sum(jnp.where(t == mb, ones, zeros))
                        return tot

                    def next_max(m_s):
                        mb = jnp.full((L,), m_s)
                        cur = neg
                        for t in ts:
                            cur = jnp.maximum(cur, jnp.where(t < mb, t, neg))
                        return jnp.max(cur)

                    m1 = jnp.max(ts[0])
                    c1 = count_eq(m1)
                    m2 = next_max(m1)
                    c2 = count_eq(m2)
                    m3 = next_max(m2)
                    c3 = count_eq(m3)
                    m4 = next_max(m3)
                    four = jnp.float32(4.0)
                    xcut = jnp.where(
                        c1 >= four, m1,
                        jnp.where(c1 + c2 >= four, m2,
                                  jnp.where(c1 + c2 + c3 >= four, m3, m4)))
                    xcbuf[pl.ds(rr * L, L)] = jnp.full((L,), xcut)

                    mnv = jnp.full((L,), jnp.min(rmin))
                    widx = (bi * blk + rr) * L
                    wb = wbuf[pl.ds(widx, L)]
                    wf = wbuf[pl.ds(rows_per_w * L + widx, L)]
                    mbuf[pl.ds(0, L)] = jnp.maximum(mbuf[pl.ds(0, L)], wb * mnv)
                    mbuf[pl.ds(L, L)] = jnp.maximum(mbuf[pl.ds(L, L)], wf * mnv)
                    return 0
                lax.fori_loop(0, blk, rowA, 0)

                for g in range(blk // G):
                    xcs, wbs, wfs = [], [], []
                    for r in range(G):
                        row = g * G + r
                        widx = (bi * blk + row) * L
                        xcs.append(xcbuf[pl.ds(row * L, L)])
                        wbs.append(wbuf[pl.ds(widx, L)])
                        wfs.append(wbuf[pl.ds(rows_per_w * L + widx, L)])

                    def pB(jj, _):
                        for j2 in range(2):
                            j = jj * 2 + j2
                            ab = acc[pl.ds(j * L, L)]
                            af = acc[pl.ds(HW + j * L, L)]
                            for r in range(G):
                                v = rb[g * G + r, pl.ds(j * L, L)]
                                xm = jnp.where(v >= xcs[r], v, zeros)
                                ab = jnp.maximum(ab, xm * wbs[r])
                                af = jnp.maximum(af, xm * wfs[r])
                            acc[pl.ds(j * L, L)] = ab
                            acc[pl.ds(HW + j * L, L)] = af
                        return 0
                    lax.fori_loop(0, nvr // 2, pB, 0)

                @pl.when(bi + 2 < nblk)
                def _():
                    start_dma(bi + 2, p)
            return 0
        lax.fori_loop(0, nblk // 2, prev_outer, 0)

        mbv = jnp.full((L,), jnp.max(mbuf[pl.ds(0, L)]))
        mfv = jnp.full((L,), jnp.max(mbuf[pl.ds(L, L)]))

        def clamp(j, _):
            acc[pl.ds(j * L, L)] = jnp.maximum(acc[pl.ds(j * L, L)], mbv)
            acc[pl.ds(HW + j * L, L)] = jnp.maximum(
                acc[pl.ds(HW + j * L, L)], mfv)
            return 0
        lax.fori_loop(0, nvr, clamp, 0)

        for ch in range(2):
            q = b * 2 + ch
            pltpu.sync_copy(acc.at[pl.ds(ch * HW, HW)],
                            out_hbm.at[pl.ds((q * NS + k) * HW, HW)])

    return body


def _sc_reduce(B, HW, H, W):
    wpr = W // L
    mesh = plsc.VectorSubcoreMesh(core_axis_name="c", subcore_axis_name="s")

    @functools.partial(
        pl.kernel,
        out_type=jax.ShapeDtypeStruct((B, 2, H, W), jnp.float32),
        mesh=mesh,
        compiler_params=pltpu.CompilerParams(needs_layout_passes=False),
        scratch_types=[
            pltpu.VMEM((NS * HW,), jnp.float32),
            pltpu.VMEM((H, W), jnp.float32),
            pltpu.SemaphoreType.DMA,
        ],
    )
    def body(part_hbm, out_hbm, pbuf, obuf, sem):
        wid = lax.axis_index("s") * NC + lax.axis_index("c")

        @pl.when(wid < 2 * B)
        def _():
            q = wid
            bb = q // 2
            ch = q % 2
            pltpu.async_copy(
                part_hbm.at[pl.ds(q * NS * HW, NS * HW)], pbuf, sem).wait()

            def red(s, _):
                for c in range(wpr):
                    j = s * wpr + c
                    m = pbuf[pl.ds(j * L, L)]
                    for kk in range(1, NS):
                        m = jnp.maximum(m, pbuf[pl.ds(kk * HW + j * L, L)])
                    obuf[s, pl.ds(c * L, L)] = m
                return 0
            lax.fori_loop(0, H, red, 0)
            pltpu.sync_copy(obuf, out_hbm.at[bb, ch])

    return body


def _tc_global(B, HW, H, W, ch_rows):
    def body(x_ref, w_ref, o_ref):
        i = pl.program_id(1)
        x = x_ref[0]
        wv = w_ref[0]
        bg = jnp.max(x * wv[0][:, None, None], axis=0)
        fg = jnp.max(x * wv[1][:, None, None], axis=0)
        cur = jnp.stack([bg, fg], axis=0)

        @pl.when(i == 0)
        def _():
            o_ref[0] = cur

        @pl.when(i > 0)
        def _():
            o_ref[0] = jnp.maximum(o_ref[0], cur)

    return pl.pallas_call(
        body,
        grid=(B, HW // ch_rows),
        in_specs=[
            pl.BlockSpec((1, ch_rows, H, W), lambda b, i: (b, i, 0, 0)),
            pl.BlockSpec((1, 2, ch_rows), lambda b, i: (b, 0, i)),
        ],
        out_specs=pl.BlockSpec((1, 2, H, W), lambda b, i: (b, 0, 0, 0)),
        out_shape=jax.ShapeDtypeStruct((B, 2, H, W), jnp.float32),
    )


def kernel(init_sim, prev_sim, init_seg, prev_seg):
    B, HW, H, W = init_sim.shape
    rows = B * HW
    rows_per_w = rows // NW
    blk = 8

    prev3 = prev_sim.reshape(B, HW, HW)
    wall = jnp.stack([prev_seg[:, 0].reshape(rows),
                      prev_seg[:, 1].reshape(rows)], 0)
    wall = jnp.broadcast_to(wall[:, :, None], (2, rows, L)).reshape(2 * rows * L)

    part = _sc_matcher(B, HW, rows_per_w, blk)(prev3, wall)
    local = _sc_reduce(B, HW, H, W)(part)
    wglob = init_seg.reshape(B, 2, HW)
    glob = _tc_global(B, HW, H, W, 256)(init_sim, wglob)
    return jnp.concatenate([glob, local], axis=1)


# speedup vs baseline: 1.6302x; 1.3114x over previous
import functools

import jax
import jax.numpy as jnp
from jax import lax
from jax.experimental import pallas as pl
from jax.experimental.pallas import tpu as pltpu
from jax.experimental.pallas import tpu_sc as plsc

L = 16
NC = 2
NS = 16
NW = NC * NS
U = 4
G = 4


def _merge4(a, b):
    z1 = jnp.maximum(a[0], b[3])
    z2 = jnp.maximum(a[1], b[2])
    z3 = jnp.maximum(a[2], b[1])
    z4 = jnp.maximum(a[3], b[0])
    w1 = jnp.maximum(z1, z3); w3 = jnp.minimum(z1, z3)
    w2 = jnp.maximum(z2, z4); w4 = jnp.minimum(z2, z4)
    s1 = jnp.maximum(w1, w2); s2 = jnp.minimum(w1, w2)
    s3 = jnp.maximum(w3, w4); s4 = jnp.minimum(w3, w4)
    return (s1, s2, s3, s4)


def _sc_matcher(B, HW, rows_per_w, blk):
    nvr = HW // L
    nblk = rows_per_w // blk
    mesh = plsc.VectorSubcoreMesh(core_axis_name="c", subcore_axis_name="s")

    @functools.partial(
        pl.kernel,
        out_type=jax.ShapeDtypeStruct((2 * B * NS * HW,), jnp.float32),
        mesh=mesh,
        compiler_params=pltpu.CompilerParams(needs_layout_passes=False),
        scratch_types=[
            pltpu.VMEM((blk, HW), jnp.float32),
            pltpu.VMEM((blk, HW), jnp.float32),
            pltpu.VMEM((2 * HW,), jnp.float32),
            pltpu.VMEM((2 * rows_per_w * L,), jnp.float32),
            pltpu.VMEM((blk * L,), jnp.float32),
            pltpu.VMEM((2 * L,), jnp.float32),
            pltpu.SemaphoreType.DMA,
            pltpu.SemaphoreType.DMA,
        ],
    )
    def body(prev_hbm, wall_hbm,
             out_hbm, rowbuf0, rowbuf1, acc, wbuf, xcbuf, mbuf, sem0, sem1):
        rowbufs = (rowbuf0, rowbuf1)
        wid = lax.axis_index("s") * NC + lax.axis_index("c")
        b = wid // NS
        k = wid % NS
        r0 = wid * rows_per_w
        rb0 = k * rows_per_w
        sems = (sem0, sem1)
        rows = B * HW

        zeros = jnp.zeros((L,), jnp.float32)
        ones = jnp.full((L,), 1.0, jnp.float32)
        neg = jnp.full((L,), -jnp.inf, jnp.float32)
        pos = jnp.full((L,), jnp.inf, jnp.float32)

        def zero_acc(j, _):
            acc[pl.ds(j * L, L)] = zeros
            return 0
        lax.fori_loop(0, 2 * nvr, zero_acc, 0)
        mbuf[pl.ds(0, L)] = zeros
        mbuf[pl.ds(L, L)] = zeros

        for seg in range(2):
            pltpu.sync_copy(
                wall_hbm.at[pl.ds(seg * rows * L + r0 * L, rows_per_w * L)],
                wbuf.at[pl.ds(seg * rows_per_w * L, rows_per_w * L)])

        def start_dma(bi, p):
            pltpu.async_copy(
                prev_hbm.at[b, pl.ds(rb0 + bi * blk, blk)],
                rowbufs[p], sems[p])

        def wait_dma(p):
            pltpu.make_async_copy(
                prev_hbm.at[0, pl.ds(0, blk)], rowbufs[p], sems[p]).wait()

        start_dma(0, 0)
        start_dma(1, 1)

        def prev_outer(h, _):
            for p in range(2):
                bi = h * 2 + p
                wait_dma(p)
                rb = rowbufs[p]

                def rowA(rr, _):
                    def p1(jj, c):
                        ts0, ts1, ts2, ts3, rmin = c
                        tss = [ts0, ts1, ts2, ts3]
                        vs = []
                        for u in range(U):
                            v = rb[rr, pl.ds((jj * U + u) * L, L)]
                            vs.append(v)
                            t1, t2, t3, t4 = tss[u]
                            lo = jnp.minimum(t1, v); t1 = jnp.maximum(t1, v)
                            lo2 = jnp.minimum(t2, lo); t2 = jnp.maximum(t2, lo)
                            lo3 = jnp.minimum(t3, lo2); t3 = jnp.maximum(t3, lo2)
                            t4 = jnp.maximum(t4, lo3)
                            tss[u] = (t1, t2, t3, t4)
                        m01 = jnp.minimum(vs[0], vs[1])
                        m23 = jnp.minimum(vs[2], vs[3])
                        rmin = jnp.minimum(rmin, jnp.minimum(m01, m23))
                        return (tss[0], tss[1], tss[2], tss[3], rmin)

                    t0 = (neg, neg, neg, neg)
                    ts0, ts1, ts2, ts3, rmin = lax.fori_loop(
                        0, nvr // U, p1, (t0, t0, t0, t0, pos))
                    ts = _merge4(_merge4(ts0, ts1), _merge4(ts2, ts3))

                    def count_eq(m_s):
                        mb = jnp.full((L,), m_s)
                        tot = jnp.float32(0.0)
                        for t in ts:
                            tot = tot + jnp.sum(jnp.where(t == mb, ones, zeros))
                        return tot

                    def next_max(m_s):
                        mb = jnp.full((L,), m_s)
                        cur = neg
                        for t in ts:
                            cur = jnp.maximum(cur, jnp.where(t < mb, t, neg))
                        return jnp.max(cur)

                    m1 = jnp.max(ts[0])
                    c1 = count_eq(m1)
                    m2 = next_max(m1)
                    c2 = count_eq(m2)
                    m3 = next_max(m2)
                    c3 = count_eq(m3)
                    m4 = next_max(m3)
                    four = jnp.float32(4.0)
                    xcut = jnp.where(
                        c1 >= four, m1,
                        jnp.where(c1 + c2 >= four, m2,
                                  jnp.where(c1 + c2 + c3 >= four, m3, m4)))
                    xcbuf[pl.ds(rr * L, L)] = jnp.full((L,), xcut)

                    mnv = jnp.full((L,), jnp.min(rmin))
                    widx = (bi * blk + rr) * L
                    wb = wbuf[pl.ds(widx, L)]
                    wf = wbuf[pl.ds(rows_per_w * L + widx, L)]
                    mbuf[pl.ds(0, L)] = jnp.maximum(mbuf[pl.ds(0, L)], wb * mnv)
                    mbuf[pl.ds(L, L)] = jnp.maximum(mbuf[pl.ds(L, L)], wf * mnv)
                    return 0
                lax.fori_loop(0, blk, rowA, 0)

                for g in range(blk // G):
                    xcs, wbs, wfs = [], [], []
                    for r in range(G):
                        row = g * G + r
                        widx = (bi * blk + row) * L
                        xcs.append(xcbuf[pl.ds(row * L, L)])
                        wbs.append(wbuf[pl.ds(widx, L)])
                        wfs.append(wbuf[pl.ds(rows_per_w * L + widx, L)])

                    def pB(jj, _):
                        for j2 in range(2):
                            j = jj * 2 + j2
                            ab = acc[pl.ds(j * L, L)]
                            af = acc[pl.ds(HW + j * L, L)]
                            for r in range(G):
                                v = rb[g * G + r, pl.ds(j * L, L)]
                                xm = jnp.where(v >= xcs[r], v, zeros)
                                ab = jnp.maximum(ab, xm * wbs[r])
                                af = jnp.maximum(af, xm * wfs[r])
                            acc[pl.ds(j * L, L)] = ab
                            acc[pl.ds(HW + j * L, L)] = af
                        return 0
                    lax.fori_loop(0, nvr // 2, pB, 0)

                @pl.when(bi + 2 < nblk)
                def _():
                    start_dma(bi + 2, p)
            return 0
        lax.fori_loop(0, nblk // 2, prev_outer, 0)

        mbv = jnp.full((L,), jnp.max(mbuf[pl.ds(0, L)]))
        mfv = jnp.full((L,), jnp.max(mbuf[pl.ds(L, L)]))

        def clamp(j, _):
            acc[pl.ds(j * L, L)] = jnp.maximum(acc[pl.ds(j * L, L)], mbv)
            acc[pl.ds(HW + j * L, L)] = jnp.maximum(
                acc[pl.ds(HW + j * L, L)], mfv)
            return 0
        lax.fori_loop(0, nvr, clamp, 0)

        for ch in range(2):
            q = b * 2 + ch
            pltpu.sync_copy(acc.at[pl.ds(ch * HW, HW)],
                            out_hbm.at[pl.ds((q * NS + k) * HW, HW)])

    return body


def _sc_reduce(B, HW, H, W):
    wpr = W // L
    mesh = plsc.VectorSubcoreMesh(core_axis_name="c", subcore_axis_name="s")

    @functools.partial(
        pl.kernel,
        out_type=jax.ShapeDtypeStruct((B, 2, H, W), jnp.float32),
        mesh=mesh,
        compiler_params=pltpu.CompilerParams(needs_layout_passes=False),
        scratch_types=[
            pltpu.VMEM((NS * HW,), jnp.float32),
            pltpu.VMEM((H, W), jnp.float32),
            pltpu.SemaphoreType.DMA,
        ],
    )
    def body(part_hbm, out_hbm, pbuf, obuf, sem):
        wid = lax.axis_index("s") * NC + lax.axis_index("c")

        @pl.when(wid < 2 * B)
        def _():
            q = wid
            bb = q // 2
            ch = q % 2
            pltpu.async_copy(
                part_hbm.at[pl.ds(q * NS * HW, NS * HW)], pbuf, sem).wait()

            def red(s, _):
                for c in range(wpr):
                    j = s * wpr + c
                    m = pbuf[pl.ds(j * L, L)]
                    for kk in range(1, NS):
                        m = jnp.maximum(m, pbuf[pl.ds(kk * HW + j * L, L)])
                    obuf[s, pl.ds(c * L, L)] = m
                return 0
            lax.fori_loop(0, H, red, 0)
            pltpu.sync_copy(obuf, out_hbm.at[bb, ch])

    return body


def _tc_global(B, HW, ch_rows):
    def body(x_ref, w_ref, o_ref):
        i = pl.program_id(1)
        x = x_ref[0]
        wv = w_ref[0]
        bg = jnp.max(x * wv[0][:, None], axis=0)
        fg = jnp.max(x * wv[1][:, None], axis=0)
        cur = jnp.stack([bg, fg], axis=0)

        @pl.when(i == 0)
        def _():
            o_ref[0] = cur

        @pl.when(i > 0)
        def _():
            o_ref[0] = jnp.maximum(o_ref[0], cur)

    return pl.pallas_call(
        body,
        grid=(B, HW // ch_rows),
        in_specs=[
            pl.BlockSpec((1, ch_rows, HW), lambda b, i: (b, i, 0)),
            pl.BlockSpec((1, 2, ch_rows), lambda b, i: (b, 0, i)),
        ],
        out_specs=pl.BlockSpec((1, 2, HW), lambda b, i: (b, 0, 0)),
        out_shape=jax.ShapeDtypeStruct((B, 2, HW), jnp.float32),
    )


def kernel(init_sim, prev_sim, init_seg, prev_seg):
    B, HW, H, W = init_sim.shape
    rows = B * HW
    rows_per_w = rows // NW
    blk = 8

    prev3 = prev_sim.reshape(B, HW, HW)
    wall = jnp.stack([prev_seg[:, 0].reshape(rows),
                      prev_seg[:, 1].reshape(rows)], 0)
    wall = jnp.broadcast_to(wall[:, :, None], (2, rows, L)).reshape(2 * rows * L)

    part = _sc_matcher(B, HW, rows_per_w, blk)(prev3, wall)
    init3 = init_sim.reshape(B, HW, HW)
    wglob = init_seg.reshape(B, 2, HW)
    glob = _tc_global(B, HW, 256)(init3, wglob).reshape(B, 2, H, W)
    local = _sc_reduce(B, HW, H, W)(part)
    return jnp.concatenate([glob, local], axis=1)


# speedup vs baseline: 2.1263x; 1.3043x over previous
import functools

import jax
import jax.numpy as jnp
from jax import lax
from jax.experimental import pallas as pl
from jax.experimental.pallas import tpu as pltpu
from jax.experimental.pallas import tpu_sc as plsc

L = 16
NC = 2
NS = 16
NW = NC * NS
U = 4
G = 4


def _merge4(a, b):
    z1 = jnp.maximum(a[0], b[3])
    z2 = jnp.maximum(a[1], b[2])
    z3 = jnp.maximum(a[2], b[1])
    z4 = jnp.maximum(a[3], b[0])
    w1 = jnp.maximum(z1, z3); w3 = jnp.minimum(z1, z3)
    w2 = jnp.maximum(z2, z4); w4 = jnp.minimum(z2, z4)
    s1 = jnp.maximum(w1, w2); s2 = jnp.minimum(w1, w2)
    s3 = jnp.maximum(w3, w4); s4 = jnp.minimum(w3, w4)
    return (s1, s2, s3, s4)


def _sc_matcher(B, HW, rows_per_w, blk):
    nvr = HW // L
    nblk = rows_per_w // blk
    mesh = plsc.VectorSubcoreMesh(core_axis_name="c", subcore_axis_name="s")

    @functools.partial(
        pl.kernel,
        out_type=jax.ShapeDtypeStruct((2 * B * NS * HW,), jnp.float32),
        mesh=mesh,
        compiler_params=pltpu.CompilerParams(needs_layout_passes=False),
        scratch_types=[
            pltpu.VMEM((blk, HW), jnp.float32),
            pltpu.VMEM((blk, HW), jnp.float32),
            pltpu.VMEM((2 * HW,), jnp.float32),
            pltpu.VMEM((2 * rows_per_w * L,), jnp.float32),
            pltpu.VMEM((blk * L,), jnp.float32),
            pltpu.VMEM((2 * L,), jnp.float32),
            pltpu.SemaphoreType.DMA,
            pltpu.SemaphoreType.DMA,
        ],
    )
    def body(prev_hbm, wall_hbm,
             out_hbm, rowbuf0, rowbuf1, acc, wbuf, xcbuf, mbuf, sem0, sem1):
        rowbufs = (rowbuf0, rowbuf1)
        wid = lax.axis_index("s") * NC + lax.axis_index("c")
        b = wid // NS
        k = wid % NS
        r0 = wid * rows_per_w
        rb0 = k * rows_per_w
        sems = (sem0, sem1)
        rows = B * HW

        zeros = jnp.zeros((L,), jnp.float32)
        ones = jnp.full((L,), 1.0, jnp.float32)
        neg = jnp.full((L,), -jnp.inf, jnp.float32)
        pos = jnp.full((L,), jnp.inf, jnp.float32)

        def zero_acc(j, _):
            acc[pl.ds(j * L, L)] = zeros
            return 0
        lax.fori_loop(0, 2 * nvr, zero_acc, 0)
        mbuf[pl.ds(0, L)] = zeros
        mbuf[pl.ds(L, L)] = zeros

        for seg in range(2):
            pltpu.sync_copy(
                wall_hbm.at[pl.ds(seg * rows * L + r0 * L, rows_per_w * L)],
                wbuf.at[pl.ds(seg * rows_per_w * L, rows_per_w * L)])

        def start_dma(bi, p):
            pltpu.async_copy(
                prev_hbm.at[b, pl.ds(rb0 + bi * blk, blk)],
                rowbufs[p], sems[p])

        def wait_dma(p):
            pltpu.make_async_copy(
                prev_hbm.at[0, pl.ds(0, blk)], rowbufs[p], sems[p]).wait()

        start_dma(0, 0)
        start_dma(1, 1)

        def prev_outer(h, _):
            for p in range(2):
                bi = h * 2 + p
                wait_dma(p)
                rb = rowbufs[p]

                def rowA(rr, _):
                    def p1(jj, c):
                        ts0, ts1, ts2, ts3, rmin = c
                        tss = [ts0, ts1, ts2, ts3]
                        vs = []
                        for u in range(U):
                            v = rb[rr, pl.ds((jj * U + u) * L, L)]
                            vs.append(v)
                            t1, t2, t3, t4 = tss[u]
                            lo = jnp.minimum(t1, v); t1 = jnp.maximum(t1, v)
                            lo2 = jnp.minimum(t2, lo); t2 = jnp.maximum(t2, lo)
                            lo3 = jnp.minimum(t3, lo2); t3 = jnp.maximum(t3, lo2)
                            t4 = jnp.maximum(t4, lo3)
                            tss[u] = (t1, t2, t3, t4)
                        m01 = jnp.minimum(vs[0], vs[1])
                        m23 = jnp.minimum(vs[2], vs[3])
                        rmin = jnp.minimum(rmin, jnp.minimum(m01, m23))
                        return (tss[0], tss[1], tss[2], tss[3], rmin)

                    t0 = (neg, neg, neg, neg)
                    ts0, ts1, ts2, ts3, rmin = lax.fori_loop(
                        0, nvr // U, p1, (t0, t0, t0, t0, pos))
                    ts = _merge4(_merge4(ts0, ts1), _merge4(ts2, ts3))

                    def count_eq(m_s):
                        mb = jnp.full((L,), m_s)
                        tot = jnp.float32(0.0)
                        for t in ts:
                            tot = tot + jnp.sum(jnp.where(t == mb, ones, zeros))
                        return tot

                    def next_max(m_s):
                        mb = jnp.full((L,), m_s)
                        cur = neg
                        for t in ts:
                            cur = jnp.maximum(cur, jnp.where(t < mb, t, neg))
                        return jnp.max(cur)

                    m1 = jnp.max(ts[0])
                    c1 = count_eq(m1)
                    m2 = next_max(m1)
                    c2 = count_eq(m2)
                    m3 = next_max(m2)
                    c3 = count_eq(m3)
                    m4 = next_max(m3)
                    four = jnp.float32(4.0)
                    xcut = jnp.where(
                        c1 >= four, m1,
                        jnp.where(c1 + c2 >= four, m2,
                                  jnp.where(c1 + c2 + c3 >= four, m3, m4)))
                    xcbuf[pl.ds(rr * L, L)] = jnp.full((L,), xcut)

                    mnv = jnp.full((L,), jnp.min(rmin))
                    widx = (bi * blk + rr) * L
                    wb = wbuf[pl.ds(widx, L)]
                    wf = wbuf[pl.ds(rows_per_w * L + widx, L)]
                    mbuf[pl.ds(0, L)] = jnp.maximum(mbuf[pl.ds(0, L)], wb * mnv)
                    mbuf[pl.ds(L, L)] = jnp.maximum(mbuf[pl.ds(L, L)], wf * mnv)
                    return 0
                lax.fori_loop(0, blk, rowA, 0)

                for g in range(blk // G):
                    xcs, wbs, wfs = [], [], []
                    for r in range(G):
                        row = g * G + r
                        widx = (bi * blk + row) * L
                        xcs.append(xcbuf[pl.ds(row * L, L)])
                        wbs.append(wbuf[pl.ds(widx, L)])
                        wfs.append(wbuf[pl.ds(rows_per_w * L + widx, L)])

                    def pB(jj, _):
                        for j2 in range(2):
                            j = jj * 2 + j2
                            ab = acc[pl.ds(j * L, L)]
                            af = acc[pl.ds(HW + j * L, L)]
                            for r in range(G):
                                v = rb[g * G + r, pl.ds(j * L, L)]
                                xm = jnp.where(v >= xcs[r], v, zeros)
                                ab = jnp.maximum(ab, xm * wbs[r])
                                af = jnp.maximum(af, xm * wfs[r])
                            acc[pl.ds(j * L, L)] = ab
                            acc[pl.ds(HW + j * L, L)] = af
                        return 0
                    lax.fori_loop(0, nvr // 2, pB, 0)

                @pl.when(bi + 2 < nblk)
                def _():
                    start_dma(bi + 2, p)
            return 0
        lax.fori_loop(0, nblk // 2, prev_outer, 0)

        mbv = jnp.full((L,), jnp.max(mbuf[pl.ds(0, L)]))
        mfv = jnp.full((L,), jnp.max(mbuf[pl.ds(L, L)]))

        def clamp(j, _):
            acc[pl.ds(j * L, L)] = jnp.maximum(acc[pl.ds(j * L, L)], mbv)
            acc[pl.ds(HW + j * L, L)] = jnp.maximum(
                acc[pl.ds(HW + j * L, L)], mfv)
            return 0
        lax.fori_loop(0, nvr, clamp, 0)

        for ch in range(2):
            q = b * 2 + ch
            pltpu.sync_copy(acc.at[pl.ds(ch * HW, HW)],
                            out_hbm.at[pl.ds((q * NS + k) * HW, HW)])

    return body


def _sc_reduce(B, HW, H, W):
    wpr = W // L
    mesh = plsc.VectorSubcoreMesh(core_axis_name="c", subcore_axis_name="s")

    @functools.partial(
        pl.kernel,
        out_type=jax.ShapeDtypeStruct((B, 4, H, W), jnp.float32),
        mesh=mesh,
        compiler_params=pltpu.CompilerParams(needs_layout_passes=False),
        scratch_types=[
            pltpu.VMEM((NS * HW,), jnp.float32),
            pltpu.VMEM((H, W), jnp.float32),
            pltpu.SemaphoreType.DMA,
        ],
    )
    def body(part_hbm, glob_hbm, out_hbm, pbuf, obuf, sem):
        wid = lax.axis_index("s") * NC + lax.axis_index("c")

        @pl.when(wid < 4 * B)
        def _():
            bb = wid // 4
            ci = wid % 4

            @pl.when(ci < 2)
            def _():
                pltpu.async_copy(
                    glob_hbm.at[pl.ds((bb * 2 + ci) * HW, HW)],
                    pbuf.at[pl.ds(0, HW)], sem).wait()

                def cpy(s, _):
                    for c in range(wpr):
                        j = s * wpr + c
                        obuf[s, pl.ds(c * L, L)] = pbuf[pl.ds(j * L, L)]
                    return 0
                lax.fori_loop(0, H, cpy, 0)
                pltpu.sync_copy(obuf, out_hbm.at[bb, ci])

            @pl.when(ci >= 2)
            def _():
                q = bb * 2 + (ci - 2)
                pltpu.async_copy(
                    part_hbm.at[pl.ds(q * NS * HW, NS * HW)], pbuf, sem).wait()

                def red(s, _):
                    for c in range(wpr):
                        j = s * wpr + c
                        m = pbuf[pl.ds(j * L, L)]
                        for kk in range(1, NS):
                            m = jnp.maximum(m, pbuf[pl.ds(kk * HW + j * L, L)])
                        obuf[s, pl.ds(c * L, L)] = m
                    return 0
                lax.fori_loop(0, H, red, 0)
                pltpu.sync_copy(obuf, out_hbm.at[bb, ci])

    return body


def _tc_global(B, HW, ch_rows):
    def body(x_ref, w_ref, o_ref):
        i = pl.program_id(1)
        x = x_ref[0]
        wv = w_ref[0]
        bg = jnp.max(x * wv[0][:, None], axis=0)
        fg = jnp.max(x * wv[1][:, None], axis=0)
        cur = jnp.stack([bg, fg], axis=0)

        @pl.when(i == 0)
        def _():
            o_ref[0] = cur

        @pl.when(i > 0)
        def _():
            o_ref[0] = jnp.maximum(o_ref[0], cur)

    return pl.pallas_call(
        body,
        grid=(B, HW // ch_rows),
        in_specs=[
            pl.BlockSpec((1, ch_rows, HW), lambda b, i: (b, i, 0)),
            pl.BlockSpec((1, 2, ch_rows), lambda b, i: (b, 0, i)),
        ],
        out_specs=pl.BlockSpec((1, 2, HW), lambda b, i: (b, 0, 0)),
        out_shape=jax.ShapeDtypeStruct((B, 2, HW), jnp.float32),
    )


def kernel(init_sim, prev_sim, init_seg, prev_seg):
    B, HW, H, W = init_sim.shape
    rows = B * HW
    rows_per_w = rows // NW
    blk = 8

    prev3 = prev_sim.reshape(B, HW, HW)
    wall = jnp.stack([prev_seg[:, 0].reshape(rows),
                      prev_seg[:, 1].reshape(rows)], 0)
    wall = jnp.broadcast_to(wall[:, :, None], (2, rows, L)).reshape(2 * rows * L)

    part = _sc_matcher(B, HW, rows_per_w, blk)(prev3, wall)
    init3 = init_sim.reshape(B, HW, HW)
    wglob = init_seg.reshape(B, 2, HW)
    glob = _tc_global(B, HW, 256)(init3, wglob).reshape(2 * B * HW)
    return _sc_reduce(B, HW, H, W)(part, glob)
